# baseline (speedup 1.0000x reference)
import functools

import jax
import jax.numpy as jnp
from jax import lax
from jax.experimental import pallas as pl
from jax.experimental.pallas import tpu as pltpu
from jax.experimental.pallas import tpu_sc as plsc

V = 10000
DIN = 128
D1 = 16
NCLS = 40
D2 = 48
K = 4
W0 = 1.0 / 5.0

HE = 80000
NW = 32
WIN = 128
NWIN = 20
EW = WIN * NWIN
HEP = NW * EW
NPAD = HEP - HE

VR = 10240
VE = 12288
DUMMY_MASK = 2047

_HIGH = lax.Precision.HIGHEST


def _tc1_body(h_ref, w1_ref, rv_ref, q_ref, hw1_ref):
    H = h_ref[...]
    rv = rv_ref[...]
    q_ref[...] = jnp.sum(H * rv[None, :], axis=1)
    hw1_ref[...] = lax.dot_general(H, w1_ref[...], (((1,), (0,)), ((), ())),
                                   precision=_HIGH)


def _tc1(h_pad, w1, rv):
    return pl.pallas_call(
        _tc1_body,
        out_shape=[jax.ShapeDtypeStruct((VR,), jnp.float32),
                   jax.ShapeDtypeStruct((VR, D1), jnp.float32)],
    )(h_pad, w1, rv)


def _tc2_body(degp_ref, hw1_ref, ds_ref, dinv_ref, g1_ref):
    deg = jnp.sum(degp_ref[...], axis=0) + 1.0
    ds = lax.rsqrt(deg)
    dinv = 1.0 / deg
    ds_ref[...] = ds
    dinv_ref[...] = dinv
    g1_ref[0:VR, :] = W0 * ds[:, None] * hw1_ref[...]
    g1_ref[VR:VE, :] = jnp.zeros((VE - VR, D1), jnp.float32)


def _tc2(deg_part, hw1):
    return pl.pallas_call(
        _tc2_body,
        out_shape=[jax.ShapeDtypeStruct((VR,), jnp.float32),
                   jax.ShapeDtypeStruct((VR,), jnp.float32),
                   jax.ShapeDtypeStruct((VE, D1), jnp.float32)],
    )(deg_part, hw1)


def _tc3_body(t1_ref, hw1_ref, ds_ref, dinv_ref, b1_ref, w2_ref,
              hw2_ref, g2_ref):
    T = t1_ref[0] + t1_ref[1]
    ds = ds_ref[...]
    dinv = dinv_ref[...]
    pre = ds[:, None] * T + dinv[:, None] * hw1_ref[...] + b1_ref[...][None, :]
    h = jnp.maximum(pre, 0.0)
    rowid = lax.broadcasted_iota(jnp.int32, (VR, D1), 0)
    h = jnp.where(rowid < V, h, 0.0)
    hw2 = lax.dot_general(h, w2_ref[...], (((1,), (0,)), ((), ())),
                          precision=_HIGH)
    hw2_ref[...] = hw2
    g2_ref[0:VR, :] = W0 * ds[:, None] * hw2
    g2_ref[VR:VE, :] = jnp.zeros((VE - VR, D2), jnp.float32)


def _tc3(t1_part, hw1, ds, dinv, b1, w2p):
    return pl.pallas_call(
        _tc3_body,
        out_shape=[jax.ShapeDtypeStruct((VR, D2), jnp.float32),
                   jax.ShapeDtypeStruct((VE, D2), jnp.float32)],
    )(t1_part, hw1, ds, dinv, b1, w2p)


def _tc4_body(t2_ref, hw2_ref, ds_ref, dinv_ref, b2_ref, out_ref):
    T = t2_ref[0] + t2_ref[1]
    pre = (ds_ref[...][:, None] * T + dinv_ref[...][:, None] * hw2_ref[...]
           + b2_ref[...][None, :])
    o = jnp.maximum(pre, 0.0)
    logits = o[0:V, 0:NCLS]
    m = jnp.max(logits, axis=1, keepdims=True)
    e = jnp.exp(logits - m)
    lse = jnp.log(jnp.sum(e, axis=1, keepdims=True)) + m
    out_ref[...] = logits - lse


def _tc4(t2_part, hw2, ds, dinv, b2p):
    return pl.pallas_call(
        _tc4_body,
        out_shape=jax.ShapeDtypeStruct((V, NCLS), jnp.float32),
    )(t2_part, hw2, ds, dinv, b2p)


_MESH = plsc.VectorSubcoreMesh(core_axis_name="c", subcore_axis_name="s")
_SC_PARAMS = pltpu.CompilerParams(needs_layout_passes=False,
                                  use_tc_tiling_on_sc=False)


@functools.partial(
    pl.kernel,
    out_type=[jax.ShapeDtypeStruct((6, HEP), jnp.int32),
              jax.ShapeDtypeStruct((NW, VR), jnp.float32)],
    mesh=_MESH,
    compiler_params=_SC_PARAMS,
    scratch_types=[
        pltpu.VMEM((VR,), jnp.float32),
        pltpu.VMEM((K * EW,), jnp.int32),
        pltpu.VMEM((6 * EW,), jnp.int32),
        pltpu.VMEM((VR,), jnp.float32),
        pltpu.SemaphoreType.DMA,
    ],
)
def _sc_edges(hed2_hbm, q_hbm, z1_hbm, src_out, deg_out,
              q_v, vf, of, deg_l, sem):
    c = lax.axis_index("c")
    s = lax.axis_index("s")
    wid = s * 2 + c
    base = wid * EW

    pltpu.sync_copy(z1_hbm, deg_l)
    for j in range(K):
        pltpu.sync_copy(hed2_hbm.at[j, pl.ds(base, EW)],
                        vf.at[pl.ds(j * EW, EW)])
    pltpu.sync_copy(q_hbm, q_v)

    lane = lax.iota(jnp.int32, 16)

    @pl.loop(0, EW // 16)
    def _grp(g):
        off = g * 16
        sl = pl.ds(off, 16)
        v = [vf[pl.ds(j * EW + off, 16)] for j in range(K)]
        p = [plsc.load_gather(q_v, [v[j]]) for j in range(K)]
        pmax = jnp.maximum(jnp.maximum(p[0], p[1]), jnp.maximum(p[2], p[3]))
        pmin = jnp.minimum(jnp.minimum(p[0], p[1]), jnp.minimum(p[2], p[3]))
        isx = [p[j] == pmax for j in range(K)]
        isn = [p[j] == pmin for j in range(K)]
        imax = [isx[0],
                isx[1] & ~isx[0],
                isx[2] & ~(isx[0] | isx[1]),
                isx[3] & ~(isx[0] | isx[1] | isx[2])]
        imin = [isn[0],
                isn[1] & ~isn[0],
                isn[2] & ~(isn[0] | isn[1]),
                isn[3] & ~(isn[0] | isn[1] | isn[2])]
        se = jnp.where(imax[0], v[0],
                       jnp.where(imax[1], v[1],
                                 jnp.where(imax[2], v[2], v[3])))
        ie = jnp.where(imin[0], v[0],
                       jnp.where(imin[1], v[1],
                                 jnp.where(imin[2], v[2], v[3])))
        med = [(v[j] != se) & (v[j] != ie) for j in range(K)]
        medf = [med[j].astype(jnp.float32) for j in range(K)]
        nmed = medf[0] + medf[1] + medf[2] + medf[3]
        f1 = [med[0],
              med[1] & ~med[0],
              med[2] & ~(med[0] | med[1]),
              med[3] & ~(med[0] | med[1] | med[2])]
        f2_1 = med[1] & med[0]
        f2_2 = med[2] & (med[0] ^ med[1])
        f2_3 = med[3] & (med[0] ^ med[1] ^ med[2])
        has1 = med[0] | med[1] | med[2] | med[3]
        has2 = f2_1 | f2_2 | f2_3
        m1 = jnp.where(f1[0], v[0],
                       jnp.where(f1[1], v[1],
                                 jnp.where(f1[2], v[2], v[3])))
        m2 = jnp.where(f2_1, v[1], jnp.where(f2_2, v[2], v[3]))
        dummy = V + ((off + wid * 16 + lane) & DUMMY_MASK)
        dummy2 = V + ((off + wid * 16 + lane + 1024) & DUMMY_MASK)
        trash = V + ((off + wid * 16 + lane) & 127)
        trash2 = V + ((off + wid * 16 + lane + 64) & 127)
        of[sl] = se
        of[pl.ds(EW + off, 16)] = ie
        of[pl.ds(2 * EW + off, 16)] = jnp.where(has1, m1, dummy)
        of[pl.ds(3 * EW + off, 16)] = jnp.where(has1, m1, trash)
        of[pl.ds(4 * EW + off, 16)] = jnp.where(has2, m2, dummy2)
        of[pl.ds(5 * EW + off, 16)] = jnp.where(has2, m2, trash2)
        for j in range(K):
            extf = imax[j].astype(jnp.float32) + imin[j].astype(jnp.float32)
            degv = W0 * (2.0 * medf[j] + extf * (1.0 + nmed))
            plsc.addupdate_scatter(deg_l, [v[j]], degv)

    for r in range(6):
        pltpu.sync_copy(of.at[pl.ds(r * EW, EW)],
                        src_out.at[r, pl.ds(base, EW)])
    pltpu.sync_copy(deg_l, deg_out.at[wid])


def _make_scatter(d, nslot):
    nch = d // 16

    @functools.partial(
        pl.kernel,
        out_type=jax.ShapeDtypeStruct((2, VR, d), jnp.float32),
        mesh=_MESH,
        compiler_params=_SC_PARAMS,
        scratch_types=(
            [pltpu.VMEM((6, NWIN, WIN), jnp.int32)]
            + [pltpu.VMEM((WIN, d), jnp.float32)] * (4 * nslot)
            + [pltpu.VMEM_SHARED((VR, d), jnp.float32)]
            + [pltpu.SemaphoreType.DMA] * (2 * nslot)
        ),
    )
    def _scatter(g_hbm, src_hbm, z_hbm, t_out, *rest):
        idx = rest[0]
        bufs = rest[1:1 + 4 * nslot]
        t_sh = rest[1 + 4 * nslot]
        sems = rest[2 + 4 * nslot:]
        c = lax.axis_index("c")
        s = lax.axis_index("s")
        wid = s * 2 + c
        slots = tuple(tuple(bufs[4 * i:4 * i + 4]) for i in range(nslot))
        gsem = sems[:nslot]
        ssem = sems[nslot:]

        pltpu.sync_copy(z_hbm.at[pl.ds(s * 640, 640), :],
                        t_sh.at[pl.ds(s * 640, 640), :])
        plsc.subcore_barrier()

        pltpu.sync_copy(src_hbm.at[:, wid], idx)

        def gathers(w, sl):
            A, B, C, E = slots[sl]
            pltpu.async_copy(g_hbm.at[idx.at[0, w]], A, gsem[sl])
            pltpu.async_copy(g_hbm.at[idx.at[1, w]], B, gsem[sl])
            pltpu.async_copy(g_hbm.at[idx.at[2, w]], C, gsem[sl])
            pltpu.async_copy(g_hbm.at[idx.at[4, w]], E, gsem[sl])

        def wait_gathers(w, sl):
            A, B, C, E = slots[sl]
            pltpu.make_async_copy(g_hbm.at[idx.at[0, w]], A, gsem[sl]).wait()
            pltpu.make_async_copy(g_hbm.at[idx.at[1, w]], B, gsem[sl]).wait()
            pltpu.make_async_copy(g_hbm.at[idx.at[2, w]], C, gsem[sl]).wait()
            pltpu.make_async_copy(g_hbm.at[idx.at[4, w]], E, gsem[sl]).wait()

        def scatters(w, sl):
            A, B, C, E = slots[sl]
            pltpu.async_copy(A, t_sh.at[idx.at[1, w]], ssem[sl], add=True)
            pltpu.async_copy(B, t_sh.at[idx.at[0, w]], ssem[sl], add=True)
            pltpu.async_copy(E, t_sh.at[idx.at[3, w]], ssem[sl], add=True)
            pltpu.async_copy(E, t_sh.at[idx.at[5, w]], ssem[sl], add=True)

        def wait_scatters(w, sl):
            A, B, C, E = slots[sl]
            pltpu.make_async_copy(A, t_sh.at[idx.at[1, w]], ssem[sl]).wait()
            pltpu.make_async_copy(B, t_sh.at[idx.at[0, w]], ssem[sl]).wait()
            pltpu.make_async_copy(E, t_sh.at[idx.at[3, w]], ssem[sl]).wait()
            pltpu.make_async_copy(E, t_sh.at[idx.at[5, w]], ssem[sl]).wait()

        def compute(sl):
            A, B, C, E = slots[sl]

            @pl.loop(0, WIN)
            def _row(e):
                for ch in range(nch):
                    cs = pl.ds(ch * 16, 16)
                    m = C[e, cs] + E[e, cs]
                    sab = A[e, cs] + B[e, cs]
                    C[e, cs] = m
                    E[e, cs] = sab
                    A[e, cs] = A[e, cs] + m
                    B[e, cs] = B[e, cs] + m

        for sl0 in range(nslot):
            gathers(sl0, sl0)

        @pl.loop(0, NWIN // nslot)
        def _win(q):
            for sl in range(nslot):
                w = q * nslot + sl
                wait_gathers(w, sl)
                compute(sl)
                scatters(w, sl)
                wait_scatters(w, sl)

                @pl.when(w + nslot < NWIN)
                def _():
                    gathers(w + nslot, sl)

        plsc.subcore_barrier()
        pltpu.sync_copy(t_sh.at[pl.ds(s * 640, 640), :],
                        t_out.at[c, pl.ds(s * 640, 640), :])

    return _scatter


_scatter_d1 = _make_scatter(D1, 4)
_scatter_d2 = _make_scatter(D2, 2)


def kernel(H, hyperedges, rv, W1, b1, W2, b2):
    f32 = jnp.float32
    h_pad = jnp.zeros((VR, DIN), f32).at[0:V, :].set(H)
    padv = (V + (jnp.arange(NPAD, dtype=jnp.int32) % (VR - V)))[:, None]
    he_pad = jnp.concatenate(
        [hyperedges.astype(jnp.int32), jnp.broadcast_to(padv, (NPAD, K))], 0)
    hed2 = he_pad.T
    w2p = jnp.zeros((D1, D2), f32).at[:, 0:NCLS].set(W2)
    b2p = jnp.zeros((D2,), f32).at[0:NCLS].set(b2)
    z1 = jnp.zeros((VR,), f32)
    zd1 = jnp.zeros((VR, D1), f32)
    zd2 = jnp.zeros((VR, D2), f32)

    q, hw1 = _tc1(h_pad, W1, rv)
    src_all, deg_part = _sc_edges(hed2, q, z1)
    src4 = src_all.reshape(6, NW, NWIN, WIN)
    ds, dinv, g1 = _tc2(deg_part, hw1)
    t1 = _scatter_d1(g1, src4, zd1)
    hw2, g2 = _tc3(t1, hw1, ds, dinv, b1, w2p)
    t2 = _scatter_d2(g2, src4, zd2)
    return _tc4(t2, hw2, ds, dinv, b2p)


# speedup vs baseline: 1.0050x; 1.0050x over previous
import functools

import jax
import jax.numpy as jnp
from jax import lax
from jax.experimental import pallas as pl
from jax.experimental.pallas import tpu as pltpu
from jax.experimental.pallas import tpu_sc as plsc

V = 10000
DIN = 128
D1 = 16
NCLS = 40
D2 = 48
K = 4
W0 = 1.0 / 5.0

HE = 80000
NW = 32
WIN = 128
NWIN = 20
EW = WIN * NWIN
HEP = NW * EW
NPAD = HEP - HE

VR = 10240
VE = 12288
DUMMY_MASK = 2047

_HIGH = lax.Precision.HIGHEST


def _tc1_body(h_ref, w1_ref, rv_ref, q_ref, hw1_ref):
    H = h_ref[...]
    rv = rv_ref[...]
    q_ref[...] = jnp.sum(H * rv[None, :], axis=1)
    hw1_ref[...] = lax.dot_general(H, w1_ref[...], (((1,), (0,)), ((), ())),
                                   precision=_HIGH)


def _tc1(h_pad, w1, rv):
    return pl.pallas_call(
        _tc1_body,
        out_shape=[jax.ShapeDtypeStruct((VR,), jnp.float32),
                   jax.ShapeDtypeStruct((VR, D1), jnp.float32)],
    )(h_pad, w1, rv)


def _tc2_body(degp_ref, hw1_ref, ds_ref, dinv_ref, g1_ref):
    deg = jnp.sum(degp_ref[...], axis=0) + 1.0
    ds = lax.rsqrt(deg)
    dinv = 1.0 / deg
    ds_ref[...] = ds
    dinv_ref[...] = dinv
    g1_ref[0:VR, :] = W0 * ds[:, None] * hw1_ref[...]
    g1_ref[VR:VE, :] = jnp.zeros((VE - VR, D1), jnp.float32)


def _tc2(deg_part, hw1):
    return pl.pallas_call(
        _tc2_body,
        out_shape=[jax.ShapeDtypeStruct((VR,), jnp.float32),
                   jax.ShapeDtypeStruct((VR,), jnp.float32),
                   jax.ShapeDtypeStruct((VE, D1), jnp.float32)],
    )(deg_part, hw1)


def _tc3_body(t1_ref, hw1_ref, ds_ref, dinv_ref, b1_ref, w2_ref,
              hw2_ref, g2_ref):
    T = t1_ref[0] + t1_ref[1]
    ds = ds_ref[...]
    dinv = dinv_ref[...]
    pre = ds[:, None] * T + dinv[:, None] * hw1_ref[...] + b1_ref[...][None, :]
    h = jnp.maximum(pre, 0.0)
    rowid = lax.broadcasted_iota(jnp.int32, (VR, D1), 0)
    h = jnp.where(rowid < V, h, 0.0)
    hw2 = lax.dot_general(h, w2_ref[...], (((1,), (0,)), ((), ())),
                          precision=_HIGH)
    hw2_ref[...] = hw2
    g2_ref[0:VR, :] = W0 * ds[:, None] * hw2
    g2_ref[VR:VE, :] = jnp.zeros((VE - VR, D2), jnp.float32)


def _tc3(t1_part, hw1, ds, dinv, b1, w2p):
    return pl.pallas_call(
        _tc3_body,
        out_shape=[jax.ShapeDtypeStruct((VR, D2), jnp.float32),
                   jax.ShapeDtypeStruct((VE, D2), jnp.float32)],
    )(t1_part, hw1, ds, dinv, b1, w2p)


def _tc4_body(t2_ref, hw2_ref, ds_ref, dinv_ref, b2_ref, out_ref):
    T = t2_ref[0] + t2_ref[1]
    pre = (ds_ref[...][:, None] * T + dinv_ref[...][:, None] * hw2_ref[...]
           + b2_ref[...][None, :])
    o = jnp.maximum(pre, 0.0)
    logits = o[0:V, 0:NCLS]
    m = jnp.max(logits, axis=1, keepdims=True)
    e = jnp.exp(logits - m)
    lse = jnp.log(jnp.sum(e, axis=1, keepdims=True)) + m
    out_ref[...] = logits - lse


def _tc4(t2_part, hw2, ds, dinv, b2p):
    return pl.pallas_call(
        _tc4_body,
        out_shape=jax.ShapeDtypeStruct((V, NCLS), jnp.float32),
    )(t2_part, hw2, ds, dinv, b2p)


_MESH = plsc.VectorSubcoreMesh(core_axis_name="c", subcore_axis_name="s")
_SC_PARAMS = pltpu.CompilerParams(needs_layout_passes=False,
                                  use_tc_tiling_on_sc=False)


@functools.partial(
    pl.kernel,
    out_type=[jax.ShapeDtypeStruct((6, HEP), jnp.int32),
              jax.ShapeDtypeStruct((NW, VR), jnp.float32)],
    mesh=_MESH,
    compiler_params=_SC_PARAMS,
    scratch_types=[
        pltpu.VMEM((VR,), jnp.float32),
        pltpu.VMEM((K * EW,), jnp.int32),
        pltpu.VMEM((6 * EW,), jnp.int32),
        pltpu.VMEM((VR,), jnp.float32),
        pltpu.SemaphoreType.DMA,
    ],
)
def _sc_edges(hed2_hbm, q_hbm, z1_hbm, src_out, deg_out,
              q_v, vf, of, deg_l, sem):
    c = lax.axis_index("c")
    s = lax.axis_index("s")
    wid = s * 2 + c
    base = wid * EW

    pltpu.sync_copy(z1_hbm, deg_l)
    for j in range(K):
        pltpu.sync_copy(hed2_hbm.at[j, pl.ds(base, EW)],
                        vf.at[pl.ds(j * EW, EW)])
    pltpu.sync_copy(q_hbm, q_v)

    lane = lax.iota(jnp.int32, 16)

    @pl.loop(0, EW // 16)
    def _grp(g):
        off = g * 16
        sl = pl.ds(off, 16)
        v = [vf[pl.ds(j * EW + off, 16)] for j in range(K)]
        p = [plsc.load_gather(q_v, [v[j]]) for j in range(K)]
        pmax = jnp.maximum(jnp.maximum(p[0], p[1]), jnp.maximum(p[2], p[3]))
        pmin = jnp.minimum(jnp.minimum(p[0], p[1]), jnp.minimum(p[2], p[3]))
        isx = [p[j] == pmax for j in range(K)]
        isn = [p[j] == pmin for j in range(K)]
        imax = [isx[0],
                isx[1] & ~isx[0],
                isx[2] & ~(isx[0] | isx[1]),
                isx[3] & ~(isx[0] | isx[1] | isx[2])]
        imin = [isn[0],
                isn[1] & ~isn[0],
                isn[2] & ~(isn[0] | isn[1]),
                isn[3] & ~(isn[0] | isn[1] | isn[2])]
        se = jnp.where(imax[0], v[0],
                       jnp.where(imax[1], v[1],
                                 jnp.where(imax[2], v[2], v[3])))
        ie = jnp.where(imin[0], v[0],
                       jnp.where(imin[1], v[1],
                                 jnp.where(imin[2], v[2], v[3])))
        med = [(v[j] != se) & (v[j] != ie) for j in range(K)]
        medf = [med[j].astype(jnp.float32) for j in range(K)]
        nmed = medf[0] + medf[1] + medf[2] + medf[3]
        f1 = [med[0],
              med[1] & ~med[0],
              med[2] & ~(med[0] | med[1]),
              med[3] & ~(med[0] | med[1] | med[2])]
        f2_1 = med[1] & med[0]
        f2_2 = med[2] & (med[0] ^ med[1])
        f2_3 = med[3] & (med[0] ^ med[1] ^ med[2])
        has1 = med[0] | med[1] | med[2] | med[3]
        has2 = f2_1 | f2_2 | f2_3
        m1 = jnp.where(f1[0], v[0],
                       jnp.where(f1[1], v[1],
                                 jnp.where(f1[2], v[2], v[3])))
        m2 = jnp.where(f2_1, v[1], jnp.where(f2_2, v[2], v[3]))
        dummy = V + ((off + wid * 16 + lane) & DUMMY_MASK)
        dummy2 = V + ((off + wid * 16 + lane + 1024) & DUMMY_MASK)
        trash = V + ((off + wid * 16 + lane) & 127)
        trash2 = V + ((off + wid * 16 + lane + 64) & 127)
        of[sl] = se
        of[pl.ds(EW + off, 16)] = ie
        of[pl.ds(2 * EW + off, 16)] = jnp.where(has1, m1, dummy)
        of[pl.ds(3 * EW + off, 16)] = jnp.where(has1, m1, trash)
        of[pl.ds(4 * EW + off, 16)] = jnp.where(has2, m2, dummy2)
        of[pl.ds(5 * EW + off, 16)] = jnp.where(has2, m2, trash2)
        for j in range(K):
            extf = imax[j].astype(jnp.float32) + imin[j].astype(jnp.float32)
            degv = W0 * (2.0 * medf[j] + extf * (1.0 + nmed))
            plsc.addupdate_scatter(deg_l, [v[j]], degv)

    for r in range(6):
        pltpu.sync_copy(of.at[pl.ds(r * EW, EW)],
                        src_out.at[r, pl.ds(base, EW)])
    pltpu.sync_copy(deg_l, deg_out.at[wid])


def _make_scatter(d, nslot):
    nch = d // 16

    @functools.partial(
        pl.kernel,
        out_type=jax.ShapeDtypeStruct((2, VR, d), jnp.float32),
        mesh=_MESH,
        compiler_params=_SC_PARAMS,
        scratch_types=(
            [pltpu.VMEM((6, NWIN, WIN), jnp.int32)]
            + [pltpu.VMEM((WIN, d), jnp.float32)] * (4 * nslot)
            + [pltpu.VMEM_SHARED((VR, d), jnp.float32)]
            + [pltpu.SemaphoreType.DMA] * (2 * nslot)
        ),
    )
    def _scatter(g_hbm, src_hbm, z_hbm, t_out, *rest):
        idx = rest[0]
        bufs = rest[1:1 + 4 * nslot]
        t_sh = rest[1 + 4 * nslot]
        sems = rest[2 + 4 * nslot:]
        c = lax.axis_index("c")
        s = lax.axis_index("s")
        wid = s * 2 + c
        slots = tuple(tuple(bufs[4 * i:4 * i + 4]) for i in range(nslot))
        gsem = sems[:nslot]
        ssem = sems[nslot:]

        pltpu.sync_copy(z_hbm.at[pl.ds(s * 640, 640), :],
                        t_sh.at[pl.ds(s * 640, 640), :])
        plsc.subcore_barrier()

        pltpu.sync_copy(src_hbm.at[:, wid], idx)

        def gathers(w, sl):
            A, B, C, E = slots[sl]
            pltpu.async_copy(g_hbm.at[idx.at[0, w]], A, gsem[sl])
            pltpu.async_copy(g_hbm.at[idx.at[1, w]], B, gsem[sl])
            pltpu.async_copy(g_hbm.at[idx.at[2, w]], C, gsem[sl])
            pltpu.async_copy(g_hbm.at[idx.at[4, w]], E, gsem[sl])

        def wait_gathers(w, sl):
            A, B, C, E = slots[sl]
            pltpu.make_async_copy(g_hbm.at[idx.at[0, w]], A, gsem[sl]).wait()
            pltpu.make_async_copy(g_hbm.at[idx.at[1, w]], B, gsem[sl]).wait()
            pltpu.make_async_copy(g_hbm.at[idx.at[2, w]], C, gsem[sl]).wait()
            pltpu.make_async_copy(g_hbm.at[idx.at[4, w]], E, gsem[sl]).wait()

        def scatters(w, sl):
            A, B, C, E = slots[sl]
            pltpu.async_copy(A, t_sh.at[idx.at[1, w]], ssem[sl], add=True)
            pltpu.async_copy(B, t_sh.at[idx.at[0, w]], ssem[sl], add=True)
            pltpu.async_copy(E, t_sh.at[idx.at[3, w]], ssem[sl], add=True)
            pltpu.async_copy(E, t_sh.at[idx.at[5, w]], ssem[sl], add=True)

        def wait_scatters(w, sl):
            A, B, C, E = slots[sl]
            pltpu.make_async_copy(A, t_sh.at[idx.at[1, w]], ssem[sl]).wait()
            pltpu.make_async_copy(B, t_sh.at[idx.at[0, w]], ssem[sl]).wait()
            pltpu.make_async_copy(E, t_sh.at[idx.at[3, w]], ssem[sl]).wait()
            pltpu.make_async_copy(E, t_sh.at[idx.at[5, w]], ssem[sl]).wait()

        def compute(sl):
            A, B, C, E = slots[sl]

            @pl.loop(0, WIN)
            def _row(e):
                for ch in range(nch):
                    cs = pl.ds(ch * 16, 16)
                    m = C[e, cs] + E[e, cs]
                    sab = A[e, cs] + B[e, cs]
                    C[e, cs] = m
                    E[e, cs] = sab
                    A[e, cs] = A[e, cs] + m
                    B[e, cs] = B[e, cs] + m

        for sl0 in range(nslot):
            gathers(sl0, sl0)

        @pl.loop(0, NWIN // nslot)
        def _win(q):
            for sl in range(nslot):
                w = q * nslot + sl
                wait_gathers(w, sl)
                compute(sl)
                scatters(w, sl)
                wait_scatters(w, sl)

                @pl.when(w + nslot < NWIN)
                def _():
                    gathers(w + nslot, sl)

        plsc.subcore_barrier()
        pltpu.sync_copy(t_sh.at[pl.ds(s * 640, 640), :],
                        t_out.at[c, pl.ds(s * 640, 640), :])

    return _scatter


_scatter_d1 = _make_scatter(D1, 2)
_scatter_d2 = _make_scatter(D2, 2)


def kernel(H, hyperedges, rv, W1, b1, W2, b2):
    f32 = jnp.float32
    h_pad = jnp.zeros((VR, DIN), f32).at[0:V, :].set(H)
    padv = (V + (jnp.arange(NPAD, dtype=jnp.int32) % (VR - V)))[:, None]
    he_pad = jnp.concatenate(
        [hyperedges.astype(jnp.int32), jnp.broadcast_to(padv, (NPAD, K))], 0)
    hed2 = he_pad.T
    w2p = jnp.zeros((D1, D2), f32).at[:, 0:NCLS].set(W2)
    b2p = jnp.zeros((D2,), f32).at[0:NCLS].set(b2)
    z1 = jnp.zeros((VR,), f32)
    zd1 = jnp.zeros((VR, D1), f32)
    zd2 = jnp.zeros((VR, D2), f32)

    q, hw1 = _tc1(h_pad, W1, rv)
    src_all, deg_part = _sc_edges(hed2, q, z1)
    src4 = src_all.reshape(6, NW, NWIN, WIN)
    ds, dinv, g1 = _tc2(deg_part, hw1)
    t1 = _scatter_d1(g1, src4, zd1)
    hw2, g2 = _tc3(t1, hw1, ds, dinv, b1, w2p)
    t2 = _scatter_d2(g2, src4, zd2)
    return _tc4(t2, hw2, ds, dinv, b2p)
